# baseline (speedup 1.0000x reference)
import functools

import jax
import jax.numpy as jnp
from jax import lax
from jax.experimental import pallas as pl
from jax.experimental.pallas import tpu as pltpu
from jax.experimental.pallas import tpu_sc as plsc

NC = 2
NS = 16
NW = NC * NS

D = 64
GL = 100
GP = 4
CHUNK = GL * GP
NBUF = 4


@functools.partial(jax.jit, static_argnums=(2,))
def _sc_gather(table, idx, total):
    per_w = total // NW
    nch = per_w // CHUNK
    assert nch % NBUF == 0 and nch >= 2 * NBUF
    mesh = plsc.VectorSubcoreMesh(core_axis_name="c", subcore_axis_name="s")

    @functools.partial(
        pl.kernel,
        mesh=mesh,
        compiler_params=pltpu.CompilerParams(use_tc_tiling_on_sc=False),
        out_type=jax.ShapeDtypeStruct((total, D), jnp.float32),
        scratch_types=[
            pltpu.VMEM((per_w // GL, GL), jnp.int32),
            pltpu.VMEM((NBUF, CHUNK, D), jnp.float32),
        ]
        + [pltpu.SemaphoreType.DMA] * NBUF
        + [pltpu.SemaphoreType.DMA] * NBUF,
    )
    def k(table_hbm, idx_hbm, out_hbm, idx_v, rows_v, *sems):
        gsem = sems[:NBUF]
        wsem = sems[NBUF:]
        wid = lax.axis_index("s") * NC + lax.axis_index("c")
        base = wid * per_w
        pltpu.sync_copy(idx_hbm.at[wid], idx_v)

        def issue_gather(ch, b):
            for j in range(GP):
                pltpu.async_copy(
                    table_hbm.at[idx_v.at[ch * GP + j]],
                    rows_v.at[b, pl.ds(j * GL, GL)],
                    gsem[b],
                )

        def drain(sem, b):
            pltpu.make_async_copy(
                out_hbm.at[pl.ds(base, CHUNK)], rows_v.at[b], sem
            ).wait()

        for b in range(NBUF - 1):
            issue_gather(b, b)

        @pl.loop(0, nch, step=NBUF)
        def _(ch0):
            for b in range(NBUF):
                ch = ch0 + b
                nxt = ch + NBUF - 1
                nb = (b + NBUF - 1) % NBUF

                @pl.when(jnp.logical_and(nxt < nch, nxt >= NBUF))
                def _():
                    drain(wsem[nb], nb)

                @pl.when(nxt < nch)
                def _():
                    issue_gather(nxt, nb)

                drain(gsem[b], b)
                pltpu.async_copy(
                    rows_v.at[b],
                    out_hbm.at[pl.ds(base + ch * CHUNK, CHUNK)],
                    wsem[b],
                )

        for b in range(NBUF):
            drain(wsem[b], b)

    return k(table, idx)


def kernel(word_indices, word_embedding_weight):
    batch, seq = word_indices.shape
    total = batch * seq
    idx = word_indices.reshape(-1).astype(jnp.int32)
    idx = idx.reshape(NW, (total // NW) // GL, GL)
    out = _sc_gather(word_embedding_weight, idx, total)
    return out.reshape(batch, seq, D)
